# baseline (speedup 1.0000x reference)
import functools

import jax
import jax.numpy as jnp
from jax import lax
from jax.experimental import pallas as pl
from jax.experimental.pallas import tpu as pltpu
from jax.experimental.pallas import tpu_sc as plsc

EMB = 64
VOC = 1_000_000
N_POS = 16384
N_NEG = 16384

VBLK = 32768
NBLK = 31
NSTEP = 16
SIGN = 2 * NSTEP * VBLK
SROW = VBLK // 2048

NC = 2
NS = 16
L = 16
NW = NC * NS
PER_W = N_POS // NW
CH = 128
NCH = PER_W // CH


def _sigma_kernel(label_ref, win_blk, wt_a, wt_b, sig_blk, col_v):
    i = pl.program_id(0)

    @pl.when(i == 0)
    def _extract_in_row():
        lbl = label_ref[0]
        lane = lax.broadcasted_iota(jnp.int32, (1, 128), 1)
        onehot = (lane == (lbl % 128)).astype(jnp.float32)
        col_v[...] = jnp.sum(win_blk[...] * onehot, axis=1, keepdims=True)

    col = col_v[...]
    for r in range(SROW):
        strip = wt_a[:, r * 2048:(r + 1) * 2048]
        dots = jnp.sum(strip * col, axis=0, keepdims=True)
        sig_blk[r:r + 1, :] = 1.0 / (1.0 + jnp.exp(-dots))
    for r in range(SROW):
        strip = wt_b[:, r * 2048:(r + 1) * 2048]
        dots = jnp.sum(strip * col, axis=0, keepdims=True)
        sig_blk[SROW + r:SROW + r + 1, :] = 1.0 / (1.0 + jnp.exp(-dots))


def _sigma_all(label, wt_in, wt_out):
    grid_spec = pltpu.PrefetchScalarGridSpec(
        num_scalar_prefetch=1,
        grid=(NSTEP,),
        in_specs=[
            pl.BlockSpec((EMB, 128), lambda i, lbl: (0, lbl[0] // 128)),
            pl.BlockSpec(
                (EMB, VBLK),
                lambda i, lbl: (0, jnp.minimum(2 * i, NBLK - 1))),
            pl.BlockSpec(
                (EMB, VBLK),
                lambda i, lbl: (0, jnp.minimum(2 * i + 1, NBLK - 1))),
        ],
        out_specs=pl.BlockSpec((2 * SROW, 2048), lambda i, lbl: (i, 0)),
        scratch_shapes=[pltpu.VMEM((EMB, 1), jnp.float32)],
    )
    return pl.pallas_call(
        _sigma_kernel,
        grid_spec=grid_spec,
        out_shape=jax.ShapeDtypeStruct((SIGN // 2048, 2048), jnp.float32),
    )(label, wt_in, wt_out, wt_out)


def _build_gather_sum():
    mesh = plsc.VectorSubcoreMesh(core_axis_name="c", subcore_axis_name="s")

    @functools.partial(
        pl.kernel,
        mesh=mesh,
        out_type=jax.ShapeDtypeStruct((NW, L), jnp.float32),
        compiler_params=pltpu.CompilerParams(
            needs_layout_passes=False, use_tc_tiling_on_sc=False),
        scratch_types=[
            pltpu.VMEM((NCH, CH), jnp.int32),
            pltpu.VMEM((NCH, CH), jnp.int32),
            pltpu.VMEM((PER_W,), jnp.float32),
            pltpu.VMEM((PER_W,), jnp.float32),
            pltpu.VMEM((L,), jnp.float32),
            pltpu.SemaphoreType.DMA,
        ],
    )
    def gather_sum(sig_hbm, pos_idx_hbm, neg_idx_hbm, out_hbm,
                   pos_iv, neg_iv, pos_sig, neg_sig, part_v, sem):
        wid = lax.axis_index("s") * NC + lax.axis_index("c")
        base = wid * NCH
        pltpu.sync_copy(pos_idx_hbm.at[pl.ds(base, NCH)], pos_iv)
        pltpu.sync_copy(neg_idx_hbm.at[pl.ds(base, NCH)], neg_iv)

        copies = []
        for j in range(NCH):
            copies.append(pltpu.async_copy(
                sig_hbm.at[pos_iv.at[j]], pos_sig.at[pl.ds(j * CH, CH)], sem))
            copies.append(pltpu.async_copy(
                sig_hbm.at[neg_iv.at[j]], neg_sig.at[pl.ds(j * CH, CH)], sem))
        for c in copies:
            c.wait()

        def body(k, acc):
            sp = pos_sig[pl.ds(k * L, L)]
            sn = neg_sig[pl.ds(k * L, L)]
            return acc + (sp - sn)
        acc = lax.fori_loop(0, PER_W // L, body, jnp.zeros((L,), jnp.float32))
        total = jnp.sum(acc)
        part_v[...] = jnp.full((L,), total)
        pltpu.sync_copy(part_v, out_hbm.at[wid])

    return gather_sum


_GATHER_SUM = _build_gather_sum()


def kernel(input_labels, pos_labels, neg_labels, W_in, W_out):
    wt_in = W_in.T
    wt_out = W_out.T
    sig = _sigma_all(input_labels.astype(jnp.int32), wt_in, wt_out)
    sig_flat = sig.reshape(SIGN)
    pos2d = pos_labels.astype(jnp.int32).reshape(NW * NCH, CH)
    neg2d = neg_labels.astype(jnp.int32).reshape(NW * NCH, CH)
    parts = _GATHER_SUM(sig_flat, pos2d, neg2d)
    return -(jnp.sum(parts[:, 0]) + jnp.float32(N_NEG))


# speedup vs baseline: 1.0310x; 1.0310x over previous
import functools

import jax
import jax.numpy as jnp
from jax import lax
from jax.experimental import pallas as pl
from jax.experimental.pallas import tpu as pltpu
from jax.experimental.pallas import tpu_sc as plsc

EMB = 64
VOC = 1_000_000
N_POS = 16384
N_NEG = 16384

VBLK = 32768
NBLK = (VOC + VBLK - 1) // VBLK
SIGN = NBLK * VBLK
SROW = VBLK // 2048

NC = 2
NS = 16
L = 16
NW = NC * NS
PER_W = N_POS // NW
CH = 128
NCH = PER_W // CH


def _sigma_kernel(label_ref, win_blk, wt_blk, sig_blk, col_v):
    i = pl.program_id(0)

    @pl.when(i == 0)
    def _extract_in_row():
        lbl = label_ref[0]
        lane = lax.broadcasted_iota(jnp.int32, (1, 128), 1)
        onehot = (lane == (lbl % 128)).astype(jnp.float32)
        col_v[...] = jnp.sum(win_blk[...] * onehot, axis=1, keepdims=True)

    col = col_v[...]
    for r in range(SROW):
        strip = wt_blk[:, r * 2048:(r + 1) * 2048]
        dots = jnp.sum(strip * col, axis=0, keepdims=True)
        sig_blk[r:r + 1, :] = 1.0 / (1.0 + jnp.exp(-dots))


def _sigma_all(label, wt_in, wt_out):
    grid_spec = pltpu.PrefetchScalarGridSpec(
        num_scalar_prefetch=1,
        grid=(NBLK,),
        in_specs=[
            pl.BlockSpec((EMB, 128), lambda i, lbl: (0, lbl[0] // 128)),
            pl.BlockSpec((EMB, VBLK), lambda i, lbl: (0, i)),
        ],
        out_specs=pl.BlockSpec((SROW, 2048), lambda i, lbl: (i, 0)),
        scratch_shapes=[pltpu.VMEM((EMB, 1), jnp.float32)],
    )
    return pl.pallas_call(
        _sigma_kernel,
        grid_spec=grid_spec,
        out_shape=jax.ShapeDtypeStruct((SIGN // 2048, 2048), jnp.float32),
    )(label, wt_in, wt_out)


def _build_gather_sum():
    mesh = plsc.VectorSubcoreMesh(core_axis_name="c", subcore_axis_name="s")

    @functools.partial(
        pl.kernel,
        mesh=mesh,
        out_type=jax.ShapeDtypeStruct((NW, L), jnp.float32),
        compiler_params=pltpu.CompilerParams(
            needs_layout_passes=False, use_tc_tiling_on_sc=False),
        scratch_types=[
            pltpu.VMEM((NCH, CH), jnp.int32),
            pltpu.VMEM((NCH, CH), jnp.int32),
            pltpu.VMEM((PER_W,), jnp.float32),
            pltpu.VMEM((PER_W,), jnp.float32),
            pltpu.VMEM((L,), jnp.float32),
            pltpu.SemaphoreType.DMA,
        ],
    )
    def gather_sum(sig_hbm, pos_idx_hbm, neg_idx_hbm, out_hbm,
                   pos_iv, neg_iv, pos_sig, neg_sig, part_v, sem):
        wid = lax.axis_index("s") * NC + lax.axis_index("c")
        base = wid * NCH
        pltpu.sync_copy(pos_idx_hbm.at[pl.ds(base, NCH)], pos_iv)
        pltpu.sync_copy(neg_idx_hbm.at[pl.ds(base, NCH)], neg_iv)

        copies = []
        for j in range(NCH):
            copies.append(pltpu.async_copy(
                sig_hbm.at[pos_iv.at[j]], pos_sig.at[pl.ds(j * CH, CH)], sem))
            copies.append(pltpu.async_copy(
                sig_hbm.at[neg_iv.at[j]], neg_sig.at[pl.ds(j * CH, CH)], sem))
        for c in copies:
            c.wait()

        def body(k, acc):
            sp = pos_sig[pl.ds(k * L, L)]
            sn = neg_sig[pl.ds(k * L, L)]
            return acc + (sp - sn)
        acc = lax.fori_loop(0, PER_W // L, body, jnp.zeros((L,), jnp.float32))
        total = jnp.sum(acc)
        part_v[...] = jnp.full((L,), total)
        pltpu.sync_copy(part_v, out_hbm.at[wid])

    return gather_sum


_GATHER_SUM = _build_gather_sum()


def kernel(input_labels, pos_labels, neg_labels, W_in, W_out):
    wt_in = W_in.T
    wt_out = W_out.T
    sig = _sigma_all(input_labels.astype(jnp.int32), wt_in, wt_out)
    sig_flat = sig.reshape(SIGN)
    pos2d = pos_labels.astype(jnp.int32).reshape(NW * NCH, CH)
    neg2d = neg_labels.astype(jnp.int32).reshape(NW * NCH, CH)
    parts = _GATHER_SUM(sig_flat, pos2d, neg2d)
    return -(jnp.sum(parts[:, 0]) + jnp.float32(N_NEG))
